# Initial kernel scaffold; baseline (speedup 1.0000x reference)
import jax
import jax.numpy as jnp
from jax.experimental import pallas as pl


def kernel(avg_features, W, b, embed):
    raise NotImplementedError("write your pallas kernel here")


# speedup vs baseline: 1.5532x; 1.5532x over previous
import functools

import jax
import jax.numpy as jnp
from jax import lax
from jax.experimental import pallas as pl
from jax.experimental.pallas import tpu as pltpu
from jax.experimental.pallas import tpu_sc as plsc

K = 10


def _tc_head_body(a_ref, w_ref, b_ref, tags_ref, topi_ref):
    logits = jnp.dot(a_ref[...], w_ref[...], preferred_element_type=jnp.float32)
    logits = logits + b_ref[...]
    m = jnp.max(logits, axis=-1, keepdims=True)
    e = jnp.exp(logits - m)
    tags_ref[...] = e / jnp.sum(e, axis=-1, keepdims=True)
    c = logits.shape[-1]
    iota = lax.broadcasted_iota(jnp.int32, logits.shape, 1)
    cur = logits
    for j in range(K):
        mx = jnp.max(cur, axis=-1, keepdims=True)
        am = jnp.min(jnp.where(cur == mx, iota, c), axis=-1, keepdims=True)
        topi_ref[:, pl.ds(j, 1)] = am
        cur = jnp.where(iota == am, -jnp.inf, cur)


def _tc_head(feats, w, b):
    bsz, d = feats.shape
    c = w.shape[1]
    bm = 256
    return pl.pallas_call(
        _tc_head_body,
        grid=(bsz // bm,),
        in_specs=[
            pl.BlockSpec((bm, d), lambda i: (i, 0)),
            pl.BlockSpec((d, c), lambda i: (0, 0)),
            pl.BlockSpec((1, c), lambda i: (0, 0)),
        ],
        out_specs=[
            pl.BlockSpec((bm, c), lambda i: (i, 0)),
            pl.BlockSpec((bm, K), lambda i: (i, 0)),
        ],
        out_shape=[
            jax.ShapeDtypeStruct((bsz, c), jnp.float32),
            jax.ShapeDtypeStruct((bsz, K), jnp.int32),
        ],
    )(feats, w, b.reshape(1, c))


def _sc_gather(embed, idx_flat):
    n = idx_flat.shape[0]
    d = embed.shape[1]
    info = plsc.get_sparse_core_info()
    nc, ns = info.num_cores, info.num_subcores
    nw = nc * ns
    n_per_w = n // nw
    chunk = 64
    n_chunks = n_per_w // chunk

    mesh = plsc.VectorSubcoreMesh(core_axis_name="c", subcore_axis_name="s")

    @functools.partial(
        pl.kernel,
        mesh=mesh,
        out_type=jax.ShapeDtypeStruct((n, d), jnp.float32),
        scratch_types=[
            pltpu.VMEM((n_per_w,), jnp.int32),
            pltpu.VMEM((chunk, d), jnp.float32),
            pltpu.SemaphoreType.DMA,
        ],
    )
    def gather_kernel(embed_hbm, idx_hbm, out_hbm, idx_v, rows_v, sem):
        wid = lax.axis_index("s") * nc + lax.axis_index("c")
        base = wid * n_per_w
        pltpu.sync_copy(idx_hbm.at[pl.ds(base, n_per_w)], idx_v)

        def body(ci, carry):
            start = pl.multiple_of(ci * chunk, chunk)
            pltpu.async_copy(
                embed_hbm.at[idx_v.at[pl.ds(start, chunk)]], rows_v, sem
            ).wait()
            pltpu.sync_copy(rows_v, out_hbm.at[pl.ds(base + start, chunk)])
            return carry

        lax.fori_loop(0, n_chunks, body, 0)

    return gather_kernel(embed, idx_flat)


def kernel(avg_features, W, b, embed):
    tags, topi = _tc_head(avg_features, W, b)
    rows = _sc_gather(embed, topi.reshape(-1))
    return tags, rows.reshape(avg_features.shape[0], K, embed.shape[1])


# speedup vs baseline: 1.5590x; 1.0037x over previous
import functools

import jax
import jax.numpy as jnp
from jax import lax
from jax.experimental import pallas as pl
from jax.experimental.pallas import tpu as pltpu
from jax.experimental.pallas import tpu_sc as plsc

K = 10


def _tc_head_body(a_ref, w_ref, b_ref, tags_ref, topi_ref):
    logits = jnp.dot(a_ref[...], w_ref[...], preferred_element_type=jnp.float32)
    logits = logits + b_ref[...]
    m = jnp.max(logits, axis=-1, keepdims=True)
    e = jnp.exp(logits - m)
    tags_ref[...] = e / jnp.sum(e, axis=-1, keepdims=True)
    c = logits.shape[-1]
    iota = lax.broadcasted_iota(jnp.int32, logits.shape, 1)
    cur = logits
    for j in range(K):
        mx = jnp.max(cur, axis=-1, keepdims=True)
        am = jnp.min(jnp.where(cur == mx, iota, c), axis=-1, keepdims=True)
        topi_ref[:, pl.ds(j, 1)] = am
        cur = jnp.where(iota == am, -jnp.inf, cur)


def _tc_head(feats, w, b):
    bsz, d = feats.shape
    c = w.shape[1]
    bm = 256
    return pl.pallas_call(
        _tc_head_body,
        grid=(bsz // bm,),
        in_specs=[
            pl.BlockSpec((bm, d), lambda i: (i, 0)),
            pl.BlockSpec((d, c), lambda i: (0, 0)),
            pl.BlockSpec((1, c), lambda i: (0, 0)),
        ],
        out_specs=[
            pl.BlockSpec((bm, c), lambda i: (i, 0)),
            pl.BlockSpec((bm, K), lambda i: (i, 0)),
        ],
        out_shape=[
            jax.ShapeDtypeStruct((bsz, c), jnp.float32),
            jax.ShapeDtypeStruct((bsz, K), jnp.int32),
        ],
    )(feats, w, b.reshape(1, c))


def _sc_gather(embed, idx_flat):
    n = idx_flat.shape[0]
    d = embed.shape[1]
    nrow = embed.shape[0]
    info = plsc.get_sparse_core_info()
    nc, ns = info.num_cores, info.num_subcores
    nw = nc * ns
    n_per_w = n // nw
    chunk = 80
    nbuf = 2
    n_chunks = n_per_w // chunk

    mesh = plsc.VectorSubcoreMesh(core_axis_name="c", subcore_axis_name="s")

    @functools.partial(
        pl.kernel,
        mesh=mesh,
        out_type=jax.ShapeDtypeStruct((n, d), jnp.float32),
        scratch_types=[
            pltpu.VMEM((n_per_w,), jnp.int32),
            pltpu.VMEM((nbuf, chunk, d), jnp.float32),
            pltpu.SemaphoreType.DMA,
            pltpu.SemaphoreType.DMA,
            pltpu.SemaphoreType.DMA,
        ],
    )
    def gather_kernel(embed_hbm, idx_hbm, out_hbm, idx_v, rows_v,
                      gsem, wsem0, wsem1):
        sid = lax.axis_index("s")
        wid = sid * nc + lax.axis_index("c")
        base = wid * n_per_w

        pltpu.sync_copy(idx_hbm.at[pl.ds(base, n_per_w)], idx_v)
        wsems = (wsem0, wsem1)

        def pair_body(p, carry):
            for s in range(nbuf):
                c = p * nbuf + s
                start = pl.multiple_of(c * chunk, chunk)

                @pl.when(p > 0)
                def _():
                    pltpu.make_async_copy(
                        rows_v.at[s], out_hbm.at[pl.ds(0, chunk)], wsems[s]
                    ).wait()

                pltpu.async_copy(
                    embed_hbm.at[idx_v.at[pl.ds(start, chunk)]],
                    rows_v.at[s], gsem,
                ).wait()
                pltpu.async_copy(
                    rows_v.at[s], out_hbm.at[pl.ds(base + start, chunk)],
                    wsems[s],
                )
            return carry

        lax.fori_loop(0, n_chunks // nbuf, pair_body, 0)
        for s in range(nbuf):
            pltpu.make_async_copy(
                rows_v.at[s], out_hbm.at[pl.ds(0, chunk)], wsems[s]
            ).wait()

    return gather_kernel(embed, idx_flat)


def kernel(avg_features, W, b, embed):
    tags, topi = _tc_head(avg_features, W, b)
    rows = _sc_gather(embed, topi.reshape(-1))
    return tags, rows.reshape(avg_features.shape[0], K, embed.shape[1])


# speedup vs baseline: 2.1936x; 1.4071x over previous
import functools

import jax
import jax.numpy as jnp
from jax import lax
from jax.experimental import pallas as pl
from jax.experimental.pallas import tpu as pltpu
from jax.experimental.pallas import tpu_sc as plsc

K = 10


def _tc_head_body(a_ref, w_ref, b_ref, tags_ref, topi_ref):
    logits = jnp.dot(a_ref[...], w_ref[...], preferred_element_type=jnp.float32)
    logits = logits + b_ref[...]
    m = jnp.max(logits, axis=-1, keepdims=True)
    e = jnp.exp(logits - m)
    tags_ref[...] = e / jnp.sum(e, axis=-1, keepdims=True)
    c = logits.shape[-1]
    iota = lax.broadcasted_iota(jnp.int32, logits.shape, 1)
    cur = logits
    for j in range(K):
        mx = jnp.max(cur, axis=-1, keepdims=True)
        am = jnp.min(jnp.where(cur == mx, iota, c), axis=-1, keepdims=True)
        topi_ref[:, pl.ds(j, 1)] = am
        cur = jnp.where(iota == am, -jnp.inf, cur)


def _tc_head(feats, w, b):
    bsz, d = feats.shape
    c = w.shape[1]
    bm = 256
    return pl.pallas_call(
        _tc_head_body,
        grid=(bsz // bm,),
        in_specs=[
            pl.BlockSpec((bm, d), lambda i: (i, 0)),
            pl.BlockSpec((d, c), lambda i: (0, 0)),
            pl.BlockSpec((1, c), lambda i: (0, 0)),
        ],
        out_specs=[
            pl.BlockSpec((bm, c), lambda i: (i, 0)),
            pl.BlockSpec((bm, K), lambda i: (i, 0)),
        ],
        out_shape=[
            jax.ShapeDtypeStruct((bsz, c), jnp.float32),
            jax.ShapeDtypeStruct((bsz, K), jnp.int32),
        ],
    )(feats, w, b.reshape(1, c))


def _sc_gather(embed, idx_t):
    bsz = idx_t.shape[1]
    d = embed.shape[1]
    info = plsc.get_sparse_core_info()
    nc, ns = info.num_cores, info.num_subcores
    nw = nc * ns
    b_per_w = bsz // nw
    cb = 64
    nbuf = 2
    n_chunks = b_per_w // cb

    mesh = plsc.VectorSubcoreMesh(core_axis_name="c", subcore_axis_name="s")

    @functools.partial(
        pl.kernel,
        mesh=mesh,
        out_type=jax.ShapeDtypeStruct((bsz, K, d), jnp.float32),
        scratch_types=[
            pltpu.VMEM((K, b_per_w), jnp.int32),
            pltpu.VMEM((nbuf, cb, d), jnp.float32),
            pltpu.SemaphoreType.DMA,
            pltpu.SemaphoreType.DMA,
            pltpu.SemaphoreType.DMA,
        ],
    )
    def gather_kernel(embed_hbm, idx_hbm, out_hbm, idx_v, rows_v,
                      gsem, wsem0, wsem1):
        sid = lax.axis_index("s")
        wid = sid * nc + lax.axis_index("c")
        wb0 = wid * b_per_w

        pltpu.sync_copy(idx_hbm.at[:, pl.ds(wb0, b_per_w)], idx_v)
        wsems = (wsem0, wsem1)

        def chunk_body(ci, carry):
            c0 = pl.multiple_of(ci * cb, cb)
            for k in range(K):
                s = k % nbuf

                @pl.when((ci > 0) | (k >= nbuf))
                def _():
                    pltpu.make_async_copy(
                        rows_v.at[s], out_hbm.at[pl.ds(0, cb), 0], wsems[s]
                    ).wait()

                pltpu.async_copy(
                    embed_hbm.at[idx_v.at[k, pl.ds(c0, cb)]],
                    rows_v.at[s], gsem,
                ).wait()
                pltpu.async_copy(
                    rows_v.at[s], out_hbm.at[pl.ds(wb0 + c0, cb), k],
                    wsems[s],
                )
            return carry

        lax.fori_loop(0, n_chunks, chunk_body, 0)
        for s in range(nbuf):
            pltpu.make_async_copy(
                rows_v.at[s], out_hbm.at[pl.ds(0, cb), 0], wsems[s]
            ).wait()

    return gather_kernel(embed, idx_t)


def kernel(avg_features, W, b, embed):
    tags, topi = _tc_head(avg_features, W, b)
    rows = _sc_gather(embed, topi.T)
    return tags, rows
